# baseline (speedup 1.0000x reference)
import jax
import jax.numpy as jnp
from jax.experimental import pallas as pl
from jax.experimental.pallas import tpu as pltpu

_NUM_ANCHORS = 3
_NUM_CLASSES = 5
_ATTRS = 5 + _NUM_CLASSES
_STRIDE = 4.0
_ANCHOR_W = (10.0, 16.0, 33.0)
_DBLK = 4


def _decode_kernel(in_ref, out_ref):
    a = pl.program_id(1)
    dj = pl.program_id(0)
    aw = jnp.where(a == 0, _ANCHOR_W[0],
                   jnp.where(a == 1, _ANCHOR_W[1], _ANCHOR_W[2]))
    aw = aw.astype(jnp.float32)

    shape = (_DBLK, 32, 128)
    lane = jax.lax.broadcasted_iota(jnp.int32, shape, 2)
    gx = (lane % 64).astype(jnp.float32)
    gy = (2 * jax.lax.broadcasted_iota(jnp.int32, shape, 1)
          + (lane // 64)).astype(jnp.float32)
    gz = (jax.lax.broadcasted_iota(jnp.int32, shape, 0)
          + dj * _DBLK).astype(jnp.float32)
    grids = (gx, gy, gz)

    for c in range(_ATTRS):
        for b in range(4):
            xe = in_ref[b, c, :, pl.Slice(0, 32, 2), :]
            xo = in_ref[b, c, :, pl.Slice(1, 32, 2), :]
            v = jnp.concatenate([xe, xo], axis=-1)
            if c < 3:
                r = (jax.nn.sigmoid(v) + grids[c]) * _STRIDE
            elif c == 3:
                r = jnp.exp(v) * aw
            else:
                r = jax.nn.sigmoid(v)
            out_ref[c, 0, pl.Slice(b, _DBLK * 32, 4), :] = r.reshape(
                _DBLK * 32, 128)


def kernel(input):
    B = input.shape[0]
    D, H, W = input.shape[2], input.shape[3], input.shape[4]
    hw = H * W
    n = _NUM_ANCHORS * D * hw

    out = pl.pallas_call(
        _decode_kernel,
        grid=(D // _DBLK, _NUM_ANCHORS),
        in_specs=[
            pl.BlockSpec(
                (B, _ATTRS, _DBLK, H, W),
                lambda dj, a: (0, a, dj, 0, 0),
            )
        ],
        out_specs=pl.BlockSpec(
            (_ATTRS, 1, _DBLK * (hw // 128) * B, 128),
            lambda dj, a: (0, a, dj, 0),
        ),
        out_shape=jax.ShapeDtypeStruct(
            (_ATTRS, _NUM_ANCHORS, D * (hw // 128) * B, 128), jnp.float32
        ),
        compiler_params=pltpu.CompilerParams(
            dimension_semantics=("parallel", "arbitrary"),
        ),
    )(input)
    out = out.reshape(_ATTRS, _NUM_ANCHORS, D * (hw // 128), B, 128)
    return out.transpose(3, 1, 2, 4, 0).reshape(B, n, _ATTRS)


# speedup vs baseline: 1.0528x; 1.0528x over previous
import jax
import jax.numpy as jnp
from jax.experimental import pallas as pl
from jax.experimental.pallas import tpu as pltpu

_NUM_ANCHORS = 3
_NUM_CLASSES = 5
_ATTRS = 5 + _NUM_CLASSES
_STRIDE = 4.0
_ANCHOR_W = (10.0, 16.0, 33.0)
_DBLK = 8


def _decode_kernel(in_ref, out_ref):
    a = pl.program_id(1)
    dj = pl.program_id(0)
    aw = jnp.where(a == 0, _ANCHOR_W[0],
                   jnp.where(a == 1, _ANCHOR_W[1], _ANCHOR_W[2]))
    aw = aw.astype(jnp.float32)

    shape = (_DBLK, 32, 128)
    lane = jax.lax.broadcasted_iota(jnp.int32, shape, 2)
    gx = (lane % 64).astype(jnp.float32)
    gy = (2 * jax.lax.broadcasted_iota(jnp.int32, shape, 1)
          + (lane // 64)).astype(jnp.float32)
    gz = (jax.lax.broadcasted_iota(jnp.int32, shape, 0)
          + dj * _DBLK).astype(jnp.float32)
    grids = (gx, gy, gz)

    for c in range(_ATTRS):
        for b in range(4):
            xe = in_ref[b, c, :, pl.Slice(0, 32, 2), :]
            xo = in_ref[b, c, :, pl.Slice(1, 32, 2), :]
            v = jnp.concatenate([xe, xo], axis=-1)
            if c < 3:
                r = (jax.nn.sigmoid(v) + grids[c]) * _STRIDE
            elif c == 3:
                r = jnp.exp(v) * aw
            else:
                r = jax.nn.sigmoid(v)
            out_ref[c, 0, pl.Slice(b, _DBLK * 32, 4), :] = r.reshape(
                _DBLK * 32, 128)


def kernel(input):
    B = input.shape[0]
    D, H, W = input.shape[2], input.shape[3], input.shape[4]
    hw = H * W
    n = _NUM_ANCHORS * D * hw

    out = pl.pallas_call(
        _decode_kernel,
        grid=(D // _DBLK, _NUM_ANCHORS),
        in_specs=[
            pl.BlockSpec(
                (B, _ATTRS, _DBLK, H, W),
                lambda dj, a: (0, a, dj, 0, 0),
            )
        ],
        out_specs=pl.BlockSpec(
            (_ATTRS, 1, _DBLK * (hw // 128) * B, 128),
            lambda dj, a: (0, a, dj, 0),
        ),
        out_shape=jax.ShapeDtypeStruct(
            (_ATTRS, _NUM_ANCHORS, D * (hw // 128) * B, 128), jnp.float32
        ),
        compiler_params=pltpu.CompilerParams(
            dimension_semantics=("parallel", "arbitrary"),
        ),
    )(input)
    out = out.reshape(_ATTRS, _NUM_ANCHORS, D * (hw // 128), B, 128)
    return out.transpose(3, 1, 2, 4, 0).reshape(B, n, _ATTRS)
